# baseline (device time: 44299 ns/iter reference)
import jax
import jax.numpy as jnp
from jax import lax
from jax.experimental import pallas as pl
from jax.experimental.pallas import tpu as pltpu

N_DEV = 4
SQ_SH = 256
SKV_USED = 1024
HQ_SH = 8
DH = 128
D_MODEL = 1024
QBLK = 64
SCALE = 0.08838834764831843
NEG = -1e9


def _body(x_ref, wq_ref, k_ref, v_ref, wo_ref, out_ref,
          ag_ref, part_ref, stg_ref, rcv_ref, ctx_ref,
          ag_send, ag_recv, rs_send, rs_recv):
    my = lax.axis_index("i")
    left = (my + N_DEV - 1) % N_DEV
    right = (my + 1) % N_DEV
    diag = (my + 2) % N_DEV

    barrier = pltpu.get_barrier_semaphore()
    for nbr in (left, right, diag):
        pl.semaphore_signal(barrier, inc=1, device_id=(nbr,),
                            device_id_type=pl.DeviceIdType.MESH)
    pl.semaphore_wait(barrier, 3)

    def mk(src, dst, ssem, rsem, dev):
        return pltpu.make_async_remote_copy(
            src_ref=src, dst_ref=dst, send_sem=ssem, recv_sem=rsem,
            device_id=(dev,), device_id_type=pl.DeviceIdType.MESH)

    def compute_chunk(xc, origin, dst, f32_out):
        q = jnp.dot(xc, wq_ref[:], preferred_element_type=jnp.float32)
        qb = (lax.broadcasted_iota(jnp.int32, (SQ_SH, SKV_USED), 0) // QBLK
              + origin * (SQ_SH // QBLK))
        kb = lax.broadcasted_iota(jnp.int32, (SQ_SH, SKV_USED), 1) // QBLK
        neg = jnp.where(kb <= qb, 0.0, NEG)
        for h in range(HQ_SH):
            qh = q[:, h * DH:(h + 1) * DH].astype(jnp.bfloat16)
            s = lax.dot_general(qh, k_ref[:, h * DH:(h + 1) * DH],
                                (((1,), (1,)), ((), ())),
                                preferred_element_type=jnp.float32)
            w = jnp.exp(s + neg)
            denom = 1.0 / jnp.sum(w, axis=1, keepdims=True)
            ctx = jnp.dot(w.astype(jnp.bfloat16),
                          v_ref[:, h * DH:(h + 1) * DH],
                          preferred_element_type=jnp.float32)
            ctx_ref[:, h * DH:(h + 1) * DH] = (ctx * denom).astype(jnp.bfloat16)
        part = jnp.dot(ctx_ref[:], wo_ref[:],
                       preferred_element_type=jnp.float32)
        dst[:] = part if f32_out else part.astype(jnp.bfloat16)

    ag_ref[0] = x_ref[:]
    agR = mk(ag_ref.at[0], ag_ref.at[3], ag_send.at[0], ag_recv.at[0], right)
    agL = mk(ag_ref.at[0], ag_ref.at[1], ag_send.at[1], ag_recv.at[1], left)
    agD = mk(ag_ref.at[0], ag_ref.at[2], ag_send.at[2], ag_recv.at[2], diag)
    agR.start()
    agL.start()

    compute_chunk(x_ref[:], my, part_ref, True)
    agD.start()

    agR.wait_recv()
    compute_chunk(ag_ref[3], (my + 3) % N_DEV, stg_ref.at[1], False)
    rsL = mk(stg_ref.at[1], rcv_ref.at[1], rs_send.at[1], rs_recv.at[1], left)
    rsL.start()

    agL.wait_recv()
    compute_chunk(ag_ref[1], (my + 1) % N_DEV, stg_ref.at[0], False)
    rsR = mk(stg_ref.at[0], rcv_ref.at[0], rs_send.at[0], rs_recv.at[0], right)
    rsR.start()

    agD.wait_recv()
    compute_chunk(ag_ref[2], (my + 2) % N_DEV, stg_ref.at[2], False)
    rsD = mk(stg_ref.at[2], rcv_ref.at[2], rs_send.at[2], rs_recv.at[2], diag)
    rsD.start()

    rsR.wait_recv()
    rsL.wait_recv()
    rsD.wait_recv()
    out_ref[:] = ((part_ref[:] + rcv_ref[0].astype(jnp.float32))
                  + (rcv_ref[1].astype(jnp.float32)
                     + rcv_ref[2].astype(jnp.float32)))

    agR.wait_send()
    agL.wait_send()
    agD.wait_send()
    rsR.wait_send()
    rsL.wait_send()
    rsD.wait_send()


def kernel(x, Wq, K_ext, V_ext, Wo):
    i = lax.axis_index("i")
    xb = x.reshape(SQ_SH, D_MODEL).astype(jnp.bfloat16)
    wq = (Wq * SCALE).astype(jnp.bfloat16)
    wo = Wo.astype(jnp.bfloat16)
    k = lax.dynamic_slice(K_ext, (0, 0, i * HQ_SH, 0),
                          (1, SKV_USED, HQ_SH, DH))
    k = k.reshape(SKV_USED, HQ_SH * DH).astype(jnp.bfloat16)
    v = lax.dynamic_slice(V_ext, (0, 0, i * HQ_SH, 0),
                          (1, SKV_USED, HQ_SH, DH))
    v = v.reshape(SKV_USED, HQ_SH * DH).astype(jnp.bfloat16)

    out = pl.pallas_call(
        _body,
        out_shape=jax.ShapeDtypeStruct((SQ_SH, D_MODEL), jnp.float32),
        in_specs=[pl.BlockSpec(memory_space=pltpu.VMEM)] * 5,
        out_specs=pl.BlockSpec(memory_space=pltpu.VMEM),
        scratch_shapes=[
            pltpu.VMEM((N_DEV, SQ_SH, D_MODEL), jnp.bfloat16),
            pltpu.VMEM((SQ_SH, D_MODEL), jnp.float32),
            pltpu.VMEM((N_DEV - 1, SQ_SH, D_MODEL), jnp.bfloat16),
            pltpu.VMEM((N_DEV - 1, SQ_SH, D_MODEL), jnp.bfloat16),
            pltpu.VMEM((SQ_SH, D_MODEL), jnp.bfloat16),
            pltpu.SemaphoreType.DMA((N_DEV - 1,)),
            pltpu.SemaphoreType.DMA((N_DEV - 1,)),
            pltpu.SemaphoreType.DMA((N_DEV - 1,)),
            pltpu.SemaphoreType.DMA((N_DEV - 1,)),
        ],
        compiler_params=pltpu.CompilerParams(collective_id=0),
    )(xb, wq, k, v, wo)
    return out.reshape(1, SQ_SH, D_MODEL)


# device time: 35990 ns/iter; 1.2309x vs baseline; 1.2309x over previous
import jax
import jax.numpy as jnp
from jax import lax
from jax.experimental import pallas as pl
from jax.experimental.pallas import tpu as pltpu

N_DEV = 4
SQ_SH = 256
SKV_USED = 1024
HQ_SH = 8
DH = 128
D_MODEL = 1024
QBLK = 64
SCALE = 0.08838834764831843
NEG = -1e9


def _body(x_ref, wq_ref, kext_ref, vext_ref, wo_ref, out_ref,
          ag_ref, part_ref, stg_ref, rcv_ref, ctx_ref,
          k32_ref, v32_ref, kbf_ref, vbf_ref,
          ag_send, ag_recv, rs_send, rs_recv, kv_sem):
    my = lax.axis_index("i")
    left = (my + N_DEV - 1) % N_DEV
    right = (my + 1) % N_DEV
    diag = (my + 2) % N_DEV

    base = my * HQ_SH
    kcp = [pltpu.make_async_copy(
        kext_ref.at[0, pl.ds(0, SKV_USED), base + h, :],
        k32_ref.at[h], kv_sem.at[0, h]) for h in range(HQ_SH)]
    vcp = [pltpu.make_async_copy(
        vext_ref.at[0, pl.ds(0, SKV_USED), base + h, :],
        v32_ref.at[h], kv_sem.at[1, h]) for h in range(HQ_SH)]
    for c in kcp:
        c.start()
    for c in vcp:
        c.start()

    barrier = pltpu.get_barrier_semaphore()
    for nbr in (left, right, diag):
        pl.semaphore_signal(barrier, inc=1, device_id=(nbr,),
                            device_id_type=pl.DeviceIdType.MESH)
    pl.semaphore_wait(barrier, 3)

    def mk(src, dst, ssem, rsem, dev):
        return pltpu.make_async_remote_copy(
            src_ref=src, dst_ref=dst, send_sem=ssem, recv_sem=rsem,
            device_id=(dev,), device_id_type=pl.DeviceIdType.MESH)

    ag_ref[0] = x_ref[:]
    agR = mk(ag_ref.at[0], ag_ref.at[3], ag_send.at[0], ag_recv.at[0], right)
    agL = mk(ag_ref.at[0], ag_ref.at[1], ag_send.at[1], ag_recv.at[1], left)
    agD = mk(ag_ref.at[0], ag_ref.at[2], ag_send.at[2], ag_recv.at[2], diag)
    agR.start()
    agL.start()

    for h in range(HQ_SH):
        kcp[h].wait()
        kbf_ref[h] = k32_ref[h].astype(jnp.bfloat16)
        vcp[h].wait()
        vbf_ref[h] = v32_ref[h].astype(jnp.bfloat16)

    def compute_chunk(xc, origin, dst, f32_out):
        q = jnp.dot(xc, wq_ref[:], preferred_element_type=jnp.float32)
        qb = (lax.broadcasted_iota(jnp.int32, (SQ_SH, SKV_USED), 0) // QBLK
              + origin * (SQ_SH // QBLK))
        kb = lax.broadcasted_iota(jnp.int32, (SQ_SH, SKV_USED), 1) // QBLK
        neg = jnp.where(kb <= qb, 0.0, NEG)
        for h in range(HQ_SH):
            qh = q[:, h * DH:(h + 1) * DH].astype(jnp.bfloat16)
            s = lax.dot_general(qh, kbf_ref[h], (((1,), (1,)), ((), ())),
                                preferred_element_type=jnp.float32)
            w = jnp.exp(s + neg)
            denom = 1.0 / jnp.sum(w, axis=1, keepdims=True)
            ctx = jnp.dot(w.astype(jnp.bfloat16), vbf_ref[h],
                          preferred_element_type=jnp.float32)
            ctx_ref[:, h * DH:(h + 1) * DH] = (ctx * denom).astype(jnp.bfloat16)
        part = jnp.dot(ctx_ref[:], wo_ref[:],
                       preferred_element_type=jnp.float32)
        dst[:] = part if f32_out else part.astype(jnp.bfloat16)

    compute_chunk(x_ref[:], my, part_ref, True)
    agD.start()

    agR.wait_recv()
    compute_chunk(ag_ref[3], (my + 3) % N_DEV, stg_ref.at[1], False)
    rsL = mk(stg_ref.at[1], rcv_ref.at[1], rs_send.at[1], rs_recv.at[1], left)
    rsL.start()

    agL.wait_recv()
    compute_chunk(ag_ref[1], (my + 1) % N_DEV, stg_ref.at[0], False)
    rsR = mk(stg_ref.at[0], rcv_ref.at[0], rs_send.at[0], rs_recv.at[0], right)
    rsR.start()

    agD.wait_recv()
    compute_chunk(ag_ref[2], (my + 2) % N_DEV, stg_ref.at[2], False)
    rsD = mk(stg_ref.at[2], rcv_ref.at[2], rs_send.at[2], rs_recv.at[2], diag)
    rsD.start()

    rsR.wait_recv()
    rsL.wait_recv()
    rsD.wait_recv()
    out_ref[:] = ((part_ref[:] + rcv_ref[0].astype(jnp.float32))
                  + (rcv_ref[1].astype(jnp.float32)
                     + rcv_ref[2].astype(jnp.float32)))

    agR.wait_send()
    agL.wait_send()
    agD.wait_send()
    rsR.wait_send()
    rsL.wait_send()
    rsD.wait_send()


def kernel(x, Wq, K_ext, V_ext, Wo):
    xb = x.reshape(SQ_SH, D_MODEL).astype(jnp.bfloat16)
    wq = (Wq * SCALE).astype(jnp.bfloat16)
    wo = Wo.astype(jnp.bfloat16)

    out = pl.pallas_call(
        _body,
        out_shape=jax.ShapeDtypeStruct((SQ_SH, D_MODEL), jnp.float32),
        in_specs=[
            pl.BlockSpec(memory_space=pltpu.VMEM),
            pl.BlockSpec(memory_space=pltpu.VMEM),
            pl.BlockSpec(memory_space=pl.ANY),
            pl.BlockSpec(memory_space=pl.ANY),
            pl.BlockSpec(memory_space=pltpu.VMEM),
        ],
        out_specs=pl.BlockSpec(memory_space=pltpu.VMEM),
        scratch_shapes=[
            pltpu.VMEM((N_DEV, SQ_SH, D_MODEL), jnp.bfloat16),
            pltpu.VMEM((SQ_SH, D_MODEL), jnp.float32),
            pltpu.VMEM((N_DEV - 1, SQ_SH, D_MODEL), jnp.bfloat16),
            pltpu.VMEM((N_DEV - 1, SQ_SH, D_MODEL), jnp.bfloat16),
            pltpu.VMEM((SQ_SH, D_MODEL), jnp.bfloat16),
            pltpu.VMEM((HQ_SH, SKV_USED, DH), jnp.float32),
            pltpu.VMEM((HQ_SH, SKV_USED, DH), jnp.float32),
            pltpu.VMEM((HQ_SH, SKV_USED, DH), jnp.bfloat16),
            pltpu.VMEM((HQ_SH, SKV_USED, DH), jnp.bfloat16),
            pltpu.SemaphoreType.DMA((N_DEV - 1,)),
            pltpu.SemaphoreType.DMA((N_DEV - 1,)),
            pltpu.SemaphoreType.DMA((N_DEV - 1,)),
            pltpu.SemaphoreType.DMA((N_DEV - 1,)),
            pltpu.SemaphoreType.DMA((2, HQ_SH)),
        ],
        compiler_params=pltpu.CompilerParams(collective_id=0),
    )(xb, wq, K_ext, V_ext, wo)
    return out.reshape(1, SQ_SH, D_MODEL)
